# baseline (speedup 1.0000x reference)
import functools

import jax
import jax.numpy as jnp
from jax import lax
from jax.experimental import pallas as pl
from jax.experimental.pallas import tpu as pltpu
from jax.experimental.pallas import tpu_sc as plsc

_B = 4096
_F = 26
_V = 100000
_D = 16
_H = 128

_NC = 2
_NS = 16
_NW = _NC * _NS
_SPT = _B // _NW
_FP = 32
_NE = _SPT * _FP
_NCH = _NE // 128
_PH = 16
_SPP = _SPT // _PH


def _lane_bcast(vec, lane):
    return jnp.take_along_axis(
        vec, jnp.full((16,), lane, jnp.int32), axis=0,
        mode="promise_in_bounds")


def _sc_so_body(bnd_hbm, off_hbm, xv_hbm, tab_hbm, so2_hbm,
                bnd_v, off_v, xv_v, val_v, so2_v,
                sem_in, sem_out, sem_a, sem_b):
    cid = lax.axis_index("c")
    sid = lax.axis_index("s")
    w = sid * _NC + cid
    sems = (sem_a, sem_b)

    cps = [
        pltpu.async_copy(bnd_hbm.at[w], bnd_v, sem_in),
        pltpu.async_copy(off_hbm.at[w], off_v, sem_in),
        pltpu.async_copy(xv_hbm.at[w], xv_v, sem_in),
    ]
    for cp in cps:
        cp.wait()

    def fire(p):
        half = (p % 2) * 256
        out = []
        for k in range(2):
            out.append(pltpu.async_copy(
                tab_hbm.at[bnd_v.at[2 * p + k]],
                val_v.at[pl.ds(half + k * 128, 128)],
                sems[p % 2],
            ))
        return out

    iota = lax.broadcasted_iota(jnp.int32, (16,), 0)

    def make_sample_body(p):
        half = (p % 2) * 256

        def sample_body(jl, _):
            j = p * _SPP + jl
            base = j * _FP
            offa = off_v[pl.ds(base, 16)]
            offb = off_v[pl.ds(base + 16, 16)]
            xva = xv_v[pl.ds(base, 16)]
            xvb = xv_v[pl.ds(base + 16, 16)]
            s_acc = jnp.zeros((16,), jnp.float32)
            q_acc = jnp.zeros((16,), jnp.float32)
            row_base = half + jl * _FP
            for f in range(_F):
                ob = _lane_bcast(offa if f < 16 else offb, f % 16)
                xb = _lane_bcast(xva if f < 16 else xvb, f % 16)
                rowv = jnp.full((16,), row_base + f, jnp.int32)
                e = plsc.load_gather(val_v, [rowv, ob + iota]) * xb
                s_acc = s_acc + e
                q_acc = q_acc + e * e
            so2_v[pl.ds(j * _D, _D)] = (s_acc * s_acc - q_acc) * 0.5
            return _

        return sample_body

    pend = fire(0)
    for p in range(_PH):
        nxt = fire(p + 1) if p + 1 < _PH else []
        for cp in pend:
            cp.wait()
        pend = nxt
        lax.fori_loop(0, _SPP, make_sample_body(p), 0)

    pltpu.async_copy(so2_v, so2_hbm.at[w], sem_out).wait()


@functools.partial(
    pl.kernel,
    out_type=jax.ShapeDtypeStruct((_NW, _SPT * _D), jnp.float32),
    mesh=plsc.VectorSubcoreMesh(core_axis_name="c", subcore_axis_name="s"),
    compiler_params=pltpu.CompilerParams(needs_layout_passes=False),
    scratch_types=(
        pltpu.VMEM((_NCH, 128), jnp.int32),
        pltpu.VMEM((_NE,), jnp.int32),
        pltpu.VMEM((_NE,), jnp.float32),
        pltpu.VMEM((512, 128), jnp.float32),
        pltpu.VMEM((_SPT * _D,), jnp.float32),
        pltpu.SemaphoreType.DMA,
        pltpu.SemaphoreType.DMA,
        pltpu.SemaphoreType.DMA,
        pltpu.SemaphoreType.DMA,
    ),
)
def _sc_so(bnd, off, xv, tab, so2, *rest):
    _sc_so_body(bnd, off, xv, tab, so2, *rest)


def _sc_fo_body(vidx_hbm, xvt_hbm, fotab_hbm, fos_hbm,
                vidx_v, xvt_v, fo_v, fos_v, sem_in, sem_g, sem_out):
    cid = lax.axis_index("c")
    sid = lax.axis_index("s")
    w = sid * _NC + cid

    cps = [
        pltpu.async_copy(vidx_hbm.at[w], vidx_v, sem_in),
        pltpu.async_copy(xvt_hbm.at[w], xvt_v, sem_in),
    ]
    for cp in cps:
        cp.wait()

    gcps = []
    for f in range(_F):
        gcps.append(pltpu.async_copy(
            fotab_hbm.at[f].at[vidx_v.at[f]],
            fo_v.at[pl.ds(f * _SPT, _SPT)],
            sem_g,
        ))
    for cp in gcps:
        cp.wait()

    def vb_body(vb, _):
        o = vb * 16
        acc = jnp.zeros((16,), jnp.float32)
        for f in range(_F):
            acc = acc + (fo_v[pl.ds(f * _SPT + o, 16)]
                         * xvt_v[pl.ds(f * _SPT + o, 16)])
        fos_v[pl.ds(o, 16)] = acc
        return _

    lax.fori_loop(0, _SPT // 16, vb_body, 0)
    pltpu.async_copy(fos_v, fos_hbm.at[w], sem_out).wait()


@functools.partial(
    pl.kernel,
    out_type=jax.ShapeDtypeStruct((_NW, _SPT), jnp.float32),
    mesh=plsc.VectorSubcoreMesh(core_axis_name="c", subcore_axis_name="s"),
    compiler_params=pltpu.CompilerParams(use_tc_tiling_on_sc=False),
    scratch_types=(
        pltpu.VMEM((_F, _SPT), jnp.int32),
        pltpu.VMEM((_F * _SPT,), jnp.float32),
        pltpu.VMEM((_F * _SPT,), jnp.float32),
        pltpu.VMEM((_SPT,), jnp.float32),
        pltpu.SemaphoreType.DMA,
        pltpu.SemaphoreType.DMA,
        pltpu.SemaphoreType.DMA,
    ),
)
def _sc_fo(vidx, xvt, fotab, fos, *rest):
    _sc_fo_body(vidx, xvt, fotab, fos, *rest)


def _tc_body(so2_ref, fos_ref, w0_ref, b0_ref, w1_ref, b1_ref, bias_ref,
             out_ref):
    x = so2_ref[...]
    h = jnp.dot(x, w0_ref[...], preferred_element_type=jnp.float32)
    h = jnp.maximum(h + b0_ref[...], 0.0)
    h = jnp.dot(h, w1_ref[...], preferred_element_type=jnp.float32)
    h = jnp.maximum(h + b1_ref[...], 0.0)
    hsum = jnp.sum(h.reshape(_NW, _SPT, _H), axis=2)
    out_ref[...] = bias_ref[0, 0] + fos_ref[...] + hsum


def _tc_mlp(so2, fosum, W0, b0, W1, b1, bias2d):
    return pl.pallas_call(
        _tc_body,
        out_shape=jax.ShapeDtypeStruct((_NW, _SPT), jnp.float32),
        in_specs=[
            pl.BlockSpec(memory_space=pltpu.VMEM),
            pl.BlockSpec(memory_space=pltpu.VMEM),
            pl.BlockSpec(memory_space=pltpu.VMEM),
            pl.BlockSpec(memory_space=pltpu.VMEM),
            pl.BlockSpec(memory_space=pltpu.VMEM),
            pl.BlockSpec(memory_space=pltpu.VMEM),
            pl.BlockSpec(memory_space=pltpu.SMEM),
        ],
        out_specs=pl.BlockSpec(memory_space=pltpu.VMEM),
    )(so2, fosum, W0, b0, W1, b1, bias2d)


def _pad32(a):
    return jnp.pad(a, ((0, 0), (0, _FP - _F))).reshape(_NW, _NE)


def kernel(Xi, Xv, fo_emb, so_emb, W0, b0, W1, b1, b):
    idx = Xi[:, :, 0].astype(jnp.int32)
    g = idx + (jnp.arange(_F, dtype=jnp.int32) * _V)[None, :]
    bnd = _pad32(g // 8).reshape(_NW, _NCH, 128)
    off = _pad32((g % 8) * _D)
    xvp = _pad32(Xv)
    vidx = idx.reshape(_NW, _SPT, _F).transpose(0, 2, 1)
    xvt = Xv.reshape(_NW, _SPT, _F).transpose(0, 2, 1).reshape(_NW, _F * _SPT)
    tab = so_emb.reshape(_F * _V, _D).reshape(_F * _V // 8, 8 * _D)
    fotab = fo_emb[:, :, 0]
    so2 = _sc_so(bnd, off, xvp, tab)
    fosum = _sc_fo(vidx, xvt, fotab)
    out2d = _tc_mlp(so2.reshape(_B, _D), fosum, W0, b0, W1, b1,
                    jnp.reshape(b.astype(jnp.float32), (1, 1)))
    return out2d.reshape(_B)


# speedup vs baseline: 4.4728x; 4.4728x over previous
import functools

import jax
import jax.numpy as jnp
from jax import lax
from jax.experimental import pallas as pl
from jax.experimental.pallas import tpu as pltpu
from jax.experimental.pallas import tpu_sc as plsc

_B = 4096
_F = 26
_V = 100000
_D = 16
_H = 128

_NC = 2
_NS = 16
_NW = _NC * _NS
_SPT = _B // _NW
_VB = _SPT // 16


def _sc_body(idxt_hbm, xvt_hbm, sotab_hbm, fotab_hbm,
             so2_hbm, fosum_hbm,
             idx_v, xvt_v, val_v, fo_v, so2_v, fos_v,
             sem_in, sem_g, sem_f, sem_out):
    cid = lax.axis_index("c")
    sid = lax.axis_index("s")
    w = sid * _NC + cid

    cps = [
        pltpu.async_copy(idxt_hbm.at[w], idx_v, sem_in),
        pltpu.async_copy(xvt_hbm.at[w], xvt_v, sem_in),
    ]
    for cp in cps:
        cp.wait()

    gcps = []
    fcps = []
    for f in range(_F):
        for d in range(_D):
            r = f * _D + d
            gcps.append(pltpu.async_copy(
                sotab_hbm.at[r].at[idx_v.at[f]],
                val_v.at[pl.ds(r * _SPT, _SPT)],
                sem_g,
            ))
        fcps.append(pltpu.async_copy(
            fotab_hbm.at[f].at[idx_v.at[f]],
            fo_v.at[pl.ds(f * _SPT, _SPT)],
            sem_f,
        ))
    for cp in gcps:
        cp.wait()
    for cp in fcps:
        cp.wait()

    def vb_body(vb, _):
        off = vb * 16
        s_acc = [jnp.zeros((16,), jnp.float32) for _ in range(_D)]
        q_acc = [jnp.zeros((16,), jnp.float32) for _ in range(_D)]
        fo_acc = jnp.zeros((16,), jnp.float32)
        for f in range(_F):
            xv = xvt_v[pl.ds(f * _SPT + off, 16)]
            for d in range(_D):
                e = val_v[pl.ds((f * _D + d) * _SPT + off, 16)] * xv
                s_acc[d] = s_acc[d] + e
                q_acc[d] = q_acc[d] + e * e
            fo_acc = fo_acc + fo_v[pl.ds(f * _SPT + off, 16)] * xv
        for d in range(_D):
            so2_v[pl.ds(d * _SPT + off, 16)] = (
                s_acc[d] * s_acc[d] - q_acc[d]) * 0.5
        fos_v[pl.ds(off, 16)] = fo_acc
        return _

    lax.fori_loop(0, _VB, vb_body, 0)

    out_cps = [
        pltpu.async_copy(so2_v, so2_hbm.at[w], sem_out),
        pltpu.async_copy(fos_v, fosum_hbm.at[w], sem_out),
    ]
    for cp in out_cps:
        cp.wait()


@functools.partial(
    pl.kernel,
    out_type=(
        jax.ShapeDtypeStruct((_NW, _D * _SPT), jnp.float32),
        jax.ShapeDtypeStruct((_NW, _SPT), jnp.float32),
    ),
    mesh=plsc.VectorSubcoreMesh(core_axis_name="c", subcore_axis_name="s"),
    compiler_params=pltpu.CompilerParams(use_tc_tiling_on_sc=False),
    scratch_types=(
        pltpu.VMEM((_F, _SPT), jnp.int32),
        pltpu.VMEM((_F * _SPT,), jnp.float32),
        pltpu.VMEM((_F * _D * _SPT,), jnp.float32),
        pltpu.VMEM((_F * _SPT,), jnp.float32),
        pltpu.VMEM((_D * _SPT,), jnp.float32),
        pltpu.VMEM((_SPT,), jnp.float32),
        pltpu.SemaphoreType.DMA,
        pltpu.SemaphoreType.DMA,
        pltpu.SemaphoreType.DMA,
        pltpu.SemaphoreType.DMA,
    ),
)
def _sc_gather_fm(idxt, xvt, sotab, fotab, so2, fosum, *rest):
    _sc_body(idxt, xvt, sotab, fotab, so2, fosum, *rest)


def _tc_body(so3_ref, fos_ref, w0_ref, b0_ref, w1_ref, b1_ref, bias_ref,
             out_ref):
    x = so3_ref[...]
    h = lax.dot_general(x, w0_ref[...], (((1,), (0,)), ((), ())),
                        preferred_element_type=jnp.float32)
    h = jnp.maximum(h + b0_ref[...], 0.0)
    h = lax.dot_general(h, w1_ref[...], (((2,), (0,)), ((), ())),
                        preferred_element_type=jnp.float32)
    h = jnp.maximum(h + b1_ref[...], 0.0)
    out_ref[...] = bias_ref[0, 0] + fos_ref[...] + jnp.sum(h, axis=2)


def _tc_mlp(so3, fosum, W0, b0, W1, b1, bias2d):
    return pl.pallas_call(
        _tc_body,
        out_shape=jax.ShapeDtypeStruct((_NW, _SPT), jnp.float32),
        in_specs=[
            pl.BlockSpec(memory_space=pltpu.VMEM),
            pl.BlockSpec(memory_space=pltpu.VMEM),
            pl.BlockSpec(memory_space=pltpu.VMEM),
            pl.BlockSpec(memory_space=pltpu.VMEM),
            pl.BlockSpec(memory_space=pltpu.VMEM),
            pl.BlockSpec(memory_space=pltpu.VMEM),
            pl.BlockSpec(memory_space=pltpu.SMEM),
        ],
        out_specs=pl.BlockSpec(memory_space=pltpu.VMEM),
    )(so3, fosum, W0, b0, W1, b1, bias2d)


def kernel(Xi, Xv, fo_emb, so_emb, W0, b0, W1, b1, b):
    idx = Xi[:, :, 0].astype(jnp.int32)
    idxt = idx.reshape(_NW, _SPT, _F).transpose(0, 2, 1)
    xvt = Xv.reshape(_NW, _SPT, _F).transpose(0, 2, 1).reshape(_NW, _F * _SPT)
    sotab = so_emb.transpose(0, 2, 1).reshape(_F * _D, _V)
    fotab = fo_emb[:, :, 0]
    so2, fosum = _sc_gather_fm(idxt, xvt, sotab, fotab)
    so3 = so2.reshape(_NW, _D, _SPT)
    out2d = _tc_mlp(so3, fosum, W0, b0, W1, b1,
                    jnp.reshape(b.astype(jnp.float32), (1, 1)))
    return out2d.reshape(_B)


# speedup vs baseline: 4.4732x; 1.0001x over previous
import functools

import jax
import jax.numpy as jnp
from jax import lax
from jax.experimental import pallas as pl
from jax.experimental.pallas import tpu as pltpu
from jax.experimental.pallas import tpu_sc as plsc

_B = 4096
_F = 26
_V = 100000
_D = 16
_H = 128

_NC = 2
_NS = 16
_NW = _NC * _NS
_SPT = _B // _NW
_VB = _SPT // 16


def _sc_body(idxt_hbm, xvt_hbm, sotab_hbm, fotab_hbm,
             so2_hbm, fosum_hbm,
             idx_v, xvt_v, val_v, fo_v, so2_v, fos_v,
             sem_in, sem_g, sem_f, sem_out):
    cid = lax.axis_index("c")
    sid = lax.axis_index("s")
    w = sid * _NC + cid

    cps = [
        pltpu.async_copy(idxt_hbm.at[w], idx_v, sem_in),
        pltpu.async_copy(xvt_hbm.at[w], xvt_v, sem_in),
    ]
    for cp in cps:
        cp.wait()

    gcps = []
    fcps = []
    for f in range(_F):
        for d in range(_D):
            r = f * _D + d
            gcps.append(pltpu.async_copy(
                sotab_hbm.at[r].at[idx_v.at[f]],
                val_v.at[pl.ds(r * _SPT, _SPT)],
                sem_g,
            ))
        fcps.append(pltpu.async_copy(
            fotab_hbm.at[f].at[idx_v.at[f]],
            fo_v.at[pl.ds(f * _SPT, _SPT)],
            sem_f,
        ))
    for cp in gcps:
        cp.wait()
    for cp in fcps:
        cp.wait()

    def vb_body(vb, _):
        off = vb * 16
        s_acc = [jnp.zeros((16,), jnp.float32) for _ in range(_D)]
        q_acc = [jnp.zeros((16,), jnp.float32) for _ in range(_D)]
        fo_acc = jnp.zeros((16,), jnp.float32)
        for f in range(_F):
            xv = xvt_v[pl.ds(f * _SPT + off, 16)]
            for d in range(_D):
                e = val_v[pl.ds((f * _D + d) * _SPT + off, 16)] * xv
                s_acc[d] = s_acc[d] + e
                q_acc[d] = q_acc[d] + e * e
            fo_acc = fo_acc + fo_v[pl.ds(f * _SPT + off, 16)] * xv
        for d in range(_D):
            so2_v[pl.ds(d * _SPT + off, 16)] = (
                s_acc[d] * s_acc[d] - q_acc[d]) * 0.5
        fos_v[pl.ds(off, 16)] = fo_acc
        return _

    lax.fori_loop(0, _VB, vb_body, 0)

    out_cps = [
        pltpu.async_copy(so2_v, so2_hbm.at[w], sem_out),
        pltpu.async_copy(fos_v, fosum_hbm.at[w], sem_out),
    ]
    for cp in out_cps:
        cp.wait()


@functools.partial(
    pl.kernel,
    out_type=(
        jax.ShapeDtypeStruct((_NW, _D * _SPT), jnp.float32),
        jax.ShapeDtypeStruct((_NW, _SPT), jnp.float32),
    ),
    mesh=plsc.VectorSubcoreMesh(core_axis_name="c", subcore_axis_name="s"),
    compiler_params=pltpu.CompilerParams(use_tc_tiling_on_sc=False),
    scratch_types=(
        pltpu.VMEM((_F, _SPT), jnp.int32),
        pltpu.VMEM((_F * _SPT,), jnp.float32),
        pltpu.VMEM((_F * _D * _SPT,), jnp.float32),
        pltpu.VMEM((_F * _SPT,), jnp.float32),
        pltpu.VMEM((_D * _SPT,), jnp.float32),
        pltpu.VMEM((_SPT,), jnp.float32),
        pltpu.SemaphoreType.DMA,
        pltpu.SemaphoreType.DMA,
        pltpu.SemaphoreType.DMA,
        pltpu.SemaphoreType.DMA,
    ),
)
def _sc_gather_fm(idxt, xvt, sotab, fotab, so2, fosum, *rest):
    _sc_body(idxt, xvt, sotab, fotab, so2, fosum, *rest)


def _tc_body(so3_ref, fos_ref, w0_ref, b0_ref, w1_ref, b1_ref, bias_ref,
             out_ref):
    x = so3_ref[...].transpose(0, 2, 1).reshape(_B, _D)
    h = jnp.dot(x, w0_ref[...], preferred_element_type=jnp.float32)
    h = jnp.maximum(h + b0_ref[...], 0.0)
    h = jnp.dot(h, w1_ref[...], preferred_element_type=jnp.float32)
    h = jnp.maximum(h + b1_ref[...], 0.0)
    hsum = jnp.sum(h.reshape(_NW, _SPT, _H), axis=2)
    out_ref[...] = bias_ref[0, 0] + fos_ref[...] + hsum


def _tc_mlp(so3, fosum, W0, b0, W1, b1, bias2d):
    return pl.pallas_call(
        _tc_body,
        out_shape=jax.ShapeDtypeStruct((_NW, _SPT), jnp.float32),
        in_specs=[
            pl.BlockSpec(memory_space=pltpu.VMEM),
            pl.BlockSpec(memory_space=pltpu.VMEM),
            pl.BlockSpec(memory_space=pltpu.VMEM),
            pl.BlockSpec(memory_space=pltpu.VMEM),
            pl.BlockSpec(memory_space=pltpu.VMEM),
            pl.BlockSpec(memory_space=pltpu.VMEM),
            pl.BlockSpec(memory_space=pltpu.SMEM),
        ],
        out_specs=pl.BlockSpec(memory_space=pltpu.VMEM),
    )(so3, fosum, W0, b0, W1, b1, bias2d)


def kernel(Xi, Xv, fo_emb, so_emb, W0, b0, W1, b1, b):
    idx = Xi[:, :, 0].astype(jnp.int32)
    idxt = idx.reshape(_NW, _SPT, _F).transpose(0, 2, 1)
    xvt = Xv.reshape(_NW, _SPT, _F).transpose(0, 2, 1).reshape(_NW, _F * _SPT)
    sotab = so_emb.transpose(0, 2, 1).reshape(_F * _D, _V)
    fotab = fo_emb.reshape(_F, _V)
    so2, fosum = _sc_gather_fm(idxt, xvt, sotab, fotab)
    so3 = so2.reshape(_NW, _D, _SPT)
    out2d = _tc_mlp(so3, fosum, W0, b0, W1, b1,
                    jnp.reshape(b.astype(jnp.float32), (1, 1)))
    return out2d.reshape(_B)
